# baseline (speedup 1.0000x reference)
import functools

import jax
import jax.numpy as jnp
from jax import lax
from jax.experimental import pallas as pl
from jax.experimental.pallas import tpu as pltpu
from jax.experimental.pallas import tpu_sc as plsc

D = 100
DP = 128
NT = 16
NSTATE = 16
NC, NS = 2, 16
NW = NC * NS
NP = 10240
HALF = NP
BLK = 512
_INTERPRET = False


def _sc_gather(table, idx, name):
    B = idx.shape[0]
    per_w = B // NW
    mesh = plsc.VectorSubcoreMesh(core_axis_name="c", subcore_axis_name="s")
    packed = table.dtype == jnp.bfloat16
    if packed:
        table = lax.bitcast_convert_type(
            table.reshape(table.shape[0], DP // 2, 2), jnp.int32)

    def _unpack(out):
        if packed:
            return lax.bitcast_convert_type(out, jnp.bfloat16).reshape(B, DP)
        return out
    W = table.shape[1]

    if per_w <= 128:
        @functools.partial(
            pl.kernel,
            mesh=mesh,
            out_type=jax.ShapeDtypeStruct((B, W), table.dtype),
            scratch_types=[
                pltpu.VMEM((per_w,), jnp.int32),
                pltpu.VMEM((per_w, W), table.dtype),
                pltpu.SemaphoreType.DMA,
            ],
            name=name,
        )
        def k(table_hbm, idx_hbm, out_hbm, idx_v, rows_v, sem):
            wid = lax.axis_index("s") * NC + lax.axis_index("c")
            base = wid * per_w
            pltpu.sync_copy(idx_hbm.at[pl.ds(base, per_w)], idx_v)
            pltpu.async_copy(table_hbm.at[idx_v], rows_v, sem).wait()
            pltpu.sync_copy(rows_v, out_hbm.at[pl.ds(base, per_w)])

        return _unpack(k(table, idx))

    assert per_w % 128 == 0
    nrow = per_w // 128
    nch = nrow
    nrow_p = (nrow + 7) // 8 * 8
    idx2d = jnp.zeros((NW, nrow_p, 128), jnp.int32).at[:, :nrow].set(
        idx.reshape(NW, nrow, 128)).reshape(NW * nrow_p, 128)

    @functools.partial(
        pl.kernel,
        mesh=mesh,
        out_type=jax.ShapeDtypeStruct((B, W), table.dtype),
        scratch_types=[
            pltpu.VMEM((nrow_p, 128), jnp.int32),
            pltpu.VMEM((128, W), table.dtype),
            pltpu.VMEM((128, W), table.dtype),
            pltpu.SemaphoreType.DMA,
            pltpu.SemaphoreType.DMA,
            pltpu.SemaphoreType.DMA,
            pltpu.SemaphoreType.DMA,
        ],
        name=name,
    )
    def kb(table_hbm, idx_hbm, out_hbm, idx_v, buf0, buf1, g0, g1, w0, w1):
        wid = lax.axis_index("s") * NC + lax.axis_index("c")
        base = wid * nrow
        pltpu.sync_copy(idx_hbm.at[pl.ds(wid * nrow_p, nrow_p)], idx_v)
        bufs, gsem, wsem = [buf0, buf1], [g0, g1], [w0, w1]
        gops = [None] * nch
        wops = [None] * nch

        def fire(ch):
            p = ch % 2
            gops[ch] = pltpu.async_copy(table_hbm.at[idx_v.at[ch]],
                                        bufs[p], gsem[p])

        fire(0)
        for ch in range(nch):
            p = ch % 2
            if ch + 1 < nch:
                if ch >= 1:
                    wops[ch - 1].wait()
                fire(ch + 1)
            gops[ch].wait()
            wops[ch] = pltpu.async_copy(
                bufs[p], out_hbm.at[pl.ds((base + ch) * 128, 128)], wsem[p])
        wops[nch - 2].wait()
        wops[nch - 1].wait()

    return _unpack(kb(table, idx2d))


def _sc_scatter(inp_e, dest, zeros_buf):
    E2 = dest.shape[0]
    ch = E2 // NS
    rows_per_tile = HALF // NS
    mesh = plsc.VectorSubcoreMesh(core_axis_name="c", subcore_axis_name="s")

    @functools.partial(
        pl.kernel,
        mesh=mesh,
        out_type=jax.ShapeDtypeStruct((2 * HALF, DP), jnp.float32),
        scratch_types=[
            pltpu.VMEM((ch,), jnp.int32),
            pltpu.VMEM((ch, DP), jnp.float32),
            pltpu.VMEM_SHARED((HALF + 8, DP), jnp.float32),
            pltpu.SemaphoreType.DMA,
        ],
        name="sc_scatter_add",
    )
    def k(inp_hbm, dest_hbm, zeros_hbm, out_hbm, idx_v, rows_v, acc, sem):
        c = lax.axis_index("c")
        s = lax.axis_index("s")
        base = c * HALF
        pltpu.sync_copy(zeros_hbm.at[pl.ds(0, rows_per_tile)],
                        acc.at[pl.ds(s * rows_per_tile, rows_per_tile)])

        @pl.when(s == 0)
        def _():
            pltpu.sync_copy(zeros_hbm.at[pl.ds(640, 8)], acc.at[pl.ds(HALF, 8)])

        plsc.subcore_barrier()
        pltpu.sync_copy(dest_hbm.at[pl.ds(s * ch, ch)], idx_v)
        for i in range(ch // 16):
            d = idx_v[pl.ds(i * 16, 16)]
            loc = d - base
            oob = (loc < 0) | (loc >= HALF)
            idx_v[pl.ds(i * 16, 16)] = jnp.where(oob, HALF, loc)
        pltpu.sync_copy(inp_hbm.at[pl.ds(s * ch, ch)], rows_v)
        pltpu.sync_copy(rows_v, acc.at[idx_v], add=True)
        plsc.subcore_barrier()
        pltpu.sync_copy(acc.at[pl.ds(s * rows_per_tile, rows_per_tile)],
                        out_hbm.at[pl.ds(base + s * rows_per_tile, rows_per_tile)])

    return k(inp_e, dest, zeros_buf)


def _f32dot(a, b):
    return jnp.dot(a, b, preferred_element_type=jnp.float32)


def _tc_prep(W_state, b_state2, W1, b12, W2, b22, edge_embed,
             g1wih, g1whh, g1bih2, g1bhh2, g2wih, g2whh, vis, etypes, masks):
    NBLK = NP // BLK
    n_real = vis.shape[2]

    def body(wst_r, bst_r, w1_r, b1_r, w2_r, b2_r, ee_r,
             w1i_r, w1h_r, b1i_r, b1h_r, w2i_r, w2h_r, vis_r, et_r, mk_r,
             w1a_o, ws2_o, cp_o, w2p_o, b2p_o, wih1_o, whh1_o, bih1_o,
             bhh1_o, wih2_o, whh2_o, bmat_o, visc_o, etc_o, mkc_o):
        i = pl.program_id(0)

        @pl.when(i == 0)
        def _weights():
            w1a = w1_r[:D, :]
            w1b = w1_r[D:, :]
            w1a_o[...] = jnp.zeros((DP, DP), jnp.float32)
            w1a_o[:D, :D] = w1a
            ws2_o[...] = jnp.zeros((NSTATE, DP), jnp.float32)
            ws2_o[:, :D] = _f32dot(wst_r[...], w1b)
            cp_o[...] = jnp.zeros((1, DP), jnp.float32)
            cp_o[:, :D] = _f32dot(bst_r[...], w1b) + b1_r[...]
            w2p_o[...] = jnp.zeros((DP, DP), jnp.float32)
            w2p_o[:D, :D] = w2_r[...]
            b2p_o[...] = jnp.zeros((1, DP), jnp.float32)
            b2p_o[:, :D] = b2_r[...]
            for src, dst in ((w1i_r, wih1_o), (w1h_r, whh1_o),
                             (w2i_r, wih2_o), (w2h_r, whh2_o)):
                dst[...] = jnp.zeros((DP, 3 * DP), jnp.float32)
                for g in range(3):
                    dst[:D, g * DP:g * DP + D] = src[:, g * D:(g + 1) * D]
            for src, dst in ((b1i_r, bih1_o), (b1h_r, bhh1_o)):
                dst[...] = jnp.zeros((1, 3 * DP), jnp.float32)
                for g in range(3):
                    dst[:, g * DP:g * DP + D] = src[:, g * D:(g + 1) * D]
            eet = jnp.transpose(ee_r[...].reshape(NT, D, D), (0, 2, 1))
            bmat_o[...] = jnp.zeros((NT * DP, DP), jnp.float32)
            for k in range(NT):
                bmat_o[k * DP:k * DP + D, :D] = eet[k]
            etm = et_r[...].reshape(4, -1)
            mkm = mk_r[...].reshape(4, -1)
            E = etm.shape[1]
            for b in range(2):
                for t in range(2):
                    r0 = t * 2 * E + b * E
                    etc_o[r0:r0 + E, :] = etm[2 * b + t].reshape(E, 1)
                    mkc_o[r0:r0 + E, :] = mkm[2 * b + t].reshape(E, 1)

        v = vis_r[...].reshape(4, BLK)
        col_ok = (lax.broadcasted_iota(jnp.int32, (4, BLK), 1)
                  + i * BLK) < n_real
        v = jnp.where(col_ok, v, 0.0)
        visc_o[...] = v.reshape(4, BLK, 1)

    E2 = 2 * etypes.shape[2]
    full = lambda *shape: pl.BlockSpec(shape, lambda i: tuple(0 for _ in shape))
    return pl.pallas_call(
        body,
        grid=(NBLK,),
        in_specs=[
            full(*W_state.shape), full(*b_state2.shape),
            full(*W1.shape), full(*b12.shape), full(*W2.shape),
            full(*b22.shape), full(*edge_embed.shape),
            full(*g1wih.shape), full(*g1whh.shape), full(*g1bih2.shape),
            full(*g1bhh2.shape), full(*g2wih.shape), full(*g2whh.shape),
            pl.BlockSpec((2, 2, BLK), lambda i: (0, 0, i)),
            full(*etypes.shape), full(*masks.shape),
        ],
        out_specs=[
            full(DP, DP), full(NSTATE, DP), full(1, DP),
            full(DP, DP), full(1, DP),
            full(DP, 3 * DP), full(DP, 3 * DP), full(1, 3 * DP),
            full(1, 3 * DP), full(DP, 3 * DP), full(DP, 3 * DP),
            full(NT * DP, DP),
            pl.BlockSpec((4, BLK, 1), lambda i: (0, i, 0)),
            full(2 * E2, 1), full(2 * E2, 1),
        ],
        out_shape=[
            jax.ShapeDtypeStruct((DP, DP), jnp.float32),
            jax.ShapeDtypeStruct((NSTATE, DP), jnp.float32),
            jax.ShapeDtypeStruct((1, DP), jnp.float32),
            jax.ShapeDtypeStruct((DP, DP), jnp.float32),
            jax.ShapeDtypeStruct((1, DP), jnp.float32),
            jax.ShapeDtypeStruct((DP, 3 * DP), jnp.float32),
            jax.ShapeDtypeStruct((DP, 3 * DP), jnp.float32),
            jax.ShapeDtypeStruct((1, 3 * DP), jnp.float32),
            jax.ShapeDtypeStruct((1, 3 * DP), jnp.float32),
            jax.ShapeDtypeStruct((DP, 3 * DP), jnp.float32),
            jax.ShapeDtypeStruct((DP, 3 * DP), jnp.float32),
            jax.ShapeDtypeStruct((NT * DP, DP), jnp.float32),
            jax.ShapeDtypeStruct((4, NP, 1), jnp.float32),
            jax.ShapeDtypeStruct((2 * E2, 1), jnp.int32),
            jax.ShapeDtypeStruct((2 * E2, 1), jnp.float32),
        ],
        interpret=_INTERPRET,
        name="tc_prep",
    )(W_state, b_state2, W1, b12, W2, b22, edge_embed,
      g1wih, g1whh, g1bih2, g1bhh2, g2wih, g2whh, vis, etypes, masks)


def _tc_init(name2_t0, states, w1a, ws2, cp, w2, b2, wih2):
    NBLK = NP // BLK

    def body(n0, s0, w1a_r, ws, c, w2r, b2r, wih, h0_ref):
        pre0 = (_f32dot(n0[...].astype(jnp.float32), w1a_r[...])
                + _f32dot(s0[...].reshape(BLK, NSTATE), ws[...]) + c[...])
        init0 = _f32dot(jax.nn.relu(pre0), w2r[...]) + b2r[...]
        gi = _f32dot(init0, wih[...])
        z = jax.nn.sigmoid(gi[:, DP:2 * DP])
        nn_ = jnp.tanh(gi[:, 2 * DP:])
        h0_ref[...] = (1.0 - z) * nn_

    row_spec = pl.BlockSpec((BLK, DP), lambda b, nb: (b * NBLK + nb, 0))
    return pl.pallas_call(
        body,
        grid=(2, NBLK),
        in_specs=[
            pl.BlockSpec((BLK, DP), lambda b, nb: (b * NBLK + nb, 0)),
            pl.BlockSpec((1, 1, BLK, NSTATE), lambda b, nb: (b, 0, nb, 0)),
            pl.BlockSpec((DP, DP), lambda b, nb: (0, 0)),
            pl.BlockSpec((NSTATE, DP), lambda b, nb: (0, 0)),
            pl.BlockSpec((1, DP), lambda b, nb: (0, 0)),
            pl.BlockSpec((DP, DP), lambda b, nb: (0, 0)),
            pl.BlockSpec((1, DP), lambda b, nb: (0, 0)),
            pl.BlockSpec((DP, 3 * DP), lambda b, nb: (0, 0)),
        ],
        out_specs=row_spec,
        out_shape=jax.ShapeDtypeStruct((2 * NP, DP), jnp.float32),
        interpret=_INTERPRET,
        name="tc_init",
    )(name2_t0, states, w1a, ws2, cp, w2, b2, wih2)


def _tc_edge(xe, et_cols, mk_cols, bmat, t_idx):
    E2 = xe.shape[0]
    EB = 256
    toff = t_idx * (E2 // EB)

    def body(xe_r, et_r, mk_r, b_r, out_r):
        xm = xe_r[...].astype(jnp.float32) * mk_r[...]
        xt = jnp.concatenate([xm] * NT, axis=1)
        kcol = lax.broadcasted_iota(jnp.int32, (EB, NT * DP), 1) // DP
        sel = (et_r[...] == kcol).astype(jnp.float32)
        out_r[...] = _f32dot(xt * sel, b_r[...])

    return pl.pallas_call(
        body,
        grid=(E2 // EB,),
        in_specs=[
            pl.BlockSpec((EB, DP), lambda i: (i, 0)),
            pl.BlockSpec((EB, 1), lambda i: (i + toff, 0)),
            pl.BlockSpec((EB, 1), lambda i: (i + toff, 0)),
            pl.BlockSpec((NT * DP, DP), lambda i: (0, 0)),
        ],
        out_specs=pl.BlockSpec((EB, DP), lambda i: (i, 0)),
        out_shape=jax.ShapeDtypeStruct((E2, DP), jnp.float32),
        interpret=_INTERPRET,
        name="tc_edge_matmul",
    )(xe, et_cols, mk_cols, bmat)


def _gru_gates(gi, gh, h):
    r = jax.nn.sigmoid(gi[:, :DP] + gh[:, :DP])
    z = jax.nn.sigmoid(gi[:, DP:2 * DP] + gh[:, DP:2 * DP])
    nn_ = jnp.tanh(gi[:, 2 * DP:] + r * gh[:, 2 * DP:])
    return (1.0 - z) * nn_ + z * h


def _tc_update(a, h, vis2, wih1, whh1, bih1, bhh1, t_idx, n_t, n_real,
               repr_buf=None, init_next=None, wih2=None, whh2=None):
    NBLK = NP // BLK
    bs = 2
    has_next = init_next is not None
    has_alias = repr_buf is not None

    def body(*refs):
        i = 7
        (a_r, h_r, vis_r, wih1_r, whh1_r, bih1_r, bhh1_r) = refs[:7]
        if has_alias:
            i += 1
        if has_next:
            (n1_r, s1_r, w1a_r, ws2_r, cp_r, w2p_r, b2p_r, wih2_r,
             whh2_r) = refs[i:i + 9]
            i += 9
        repr_r, glob_r = refs[i:i + 2]
        i += 2
        if has_next:
            h1_r = refs[i]
            i += 1
        accg, accv = refs[i:i + 2]
        nb = pl.program_id(1)
        h_ = h_r[...].astype(jnp.float32)
        gi = _f32dot(a_r[...], wih1_r[...]) + bih1_r[...]
        gh = _f32dot(h_, whh1_r[...]) + bhh1_r[...]
        feat = _gru_gates(gi, gh, h_)
        vis = vis_r[...].reshape(BLK, 1)
        rep = feat * vis
        repr_r[...] = rep[None, None, :, :D]
        ok = (lax.broadcasted_iota(jnp.int32, (BLK, 1), 0) + nb * BLK) < n_real
        part = jnp.sum(jnp.where(ok, rep * vis, 0.0), axis=0, keepdims=True)
        vpart = jnp.sum(vis) * jnp.ones((1, DP), jnp.float32)
        prev_g = jnp.where(nb == 0, 0.0, accg[...])
        prev_v = jnp.where(nb == 0, 0.0, accv[...])
        accg[...] = prev_g + part
        accv[...] = prev_v + vpart
        glob_r[...] = jnp.broadcast_to(
            (accg[...] / (accv[...] + 1e-9))[None], (1, 8, DP))
        if has_next:
            pre1 = (_f32dot(n1_r[...].astype(jnp.float32), w1a_r[...])
                    + _f32dot(s1_r[...].reshape(BLK, NSTATE), ws2_r[...])
                    + cp_r[...])
            init1 = _f32dot(jax.nn.relu(pre1), w2p_r[...]) + b2p_r[...]
            gi2 = _f32dot(init1, wih2_r[...])
            gh2 = _f32dot(h_, whh2_r[...])
            h1_r[...] = _gru_gates(gi2, gh2, h_)

    row_spec = pl.BlockSpec((BLK, DP), lambda b, nb: (b * NBLK + nb, 0))
    w_spec = pl.BlockSpec((DP, 3 * DP), lambda b, nb: (0, 0))
    b_spec = pl.BlockSpec((1, 3 * DP), lambda b, nb: (0, 0))
    in_specs = [row_spec, row_spec,
                pl.BlockSpec((1, BLK, 1), lambda b, nb: (2 * b + t_idx, nb, 0)),
                w_spec, w_spec, b_spec, b_spec]
    args = [a, h, vis2, wih1, whh1, bih1, bhh1]
    aliases = {}
    if has_alias:
        in_specs.append(pl.BlockSpec(memory_space=pl.ANY))
        args.append(repr_buf)
        aliases = {7: 0}
    if has_next:
        (name2_t1, states_raw, w1a, ws2, cp, w2p, b2p, wih2, whh2) = init_next
        in_specs += [
            pl.BlockSpec((BLK, DP), lambda b, nb: (b * NBLK + nb, 0)),
            pl.BlockSpec((1, 1, BLK, NSTATE), lambda b, nb: (b, 1, nb, 0)),
            pl.BlockSpec((DP, DP), lambda b, nb: (0, 0)),
            pl.BlockSpec((NSTATE, DP), lambda b, nb: (0, 0)),
            pl.BlockSpec((1, DP), lambda b, nb: (0, 0)),
            pl.BlockSpec((DP, DP), lambda b, nb: (0, 0)),
            pl.BlockSpec((1, DP), lambda b, nb: (0, 0)),
            w_spec, w_spec]
        args += [name2_t1, states_raw, w1a, ws2, cp, w2p, b2p, wih2, whh2]
    out_specs = [
        pl.BlockSpec((1, 1, BLK, D), lambda b, nb: (b, t_idx, nb, 0)),
        pl.BlockSpec((1, 8, DP), lambda b, nb: (b, 0, 0)),
    ]
    out_shape = [
        jax.ShapeDtypeStruct((bs, n_t, n_real, D), jnp.float32),
        jax.ShapeDtypeStruct((bs, 8, DP), jnp.float32),
    ]
    if has_next:
        out_specs.append(row_spec)
        out_shape.append(jax.ShapeDtypeStruct((bs * NP, DP), jnp.float32))

    return pl.pallas_call(
        body,
        grid=(bs, NBLK),
        in_specs=in_specs,
        out_specs=out_specs,
        out_shape=out_shape,
        input_output_aliases=aliases,
        scratch_shapes=[pltpu.VMEM((1, DP), jnp.float32),
                        pltpu.VMEM((1, DP), jnp.float32)],
        interpret=_INTERPRET,
        name="tc_update_t%d" % t_idx,
    )(*args)


def kernel(class_names, states, edge_values, edge_types, visibility, mask_edges,
           obj_emb, W_state, b_state, W1, b1, W2, b2, edge_embed,
           gru1_wih, gru1_whh, gru1_bih, gru1_bhh, gru2_wih, gru2_whh):
    bs, T, N = class_names.shape
    E = edge_values.shape[2]
    E2 = bs * E
    f32 = jnp.float32
    i32 = jnp.int32

    (w1a, ws2, cp, w2p, b2p, wih1, whh1, bih1, bhh1, wih2, whh2, bmat,
     vis_cols, et_cols, mk_cols) = _tc_prep(
        W_state, b_state.reshape(1, D), W1, b1.reshape(1, D), W2,
        b2.reshape(1, D), edge_embed, gru1_wih, gru1_whh,
        gru1_bih.reshape(1, 3 * D), gru1_bhh.reshape(1, 3 * D), gru2_wih,
        gru2_whh, visibility, edge_types.astype(i32), mask_edges)
    obj_pad = jnp.zeros((obj_emb.shape[0], DP), f32).at[:, :D].set(obj_emb)

    class_t = class_names.astype(i32).transpose(1, 0, 2)
    class_p = jnp.zeros((T, bs, NP), i32).at[:, :, :N].set(class_t).reshape(-1)
    offs = (jnp.arange(bs, dtype=i32) * NP)[None, :, None]
    ev = edge_values.astype(i32).transpose(1, 0, 2, 3)
    origin = (ev[:, :, :, 0] + offs).reshape(T, E2)
    dest = (ev[:, :, :, 1] + offs).reshape(T, E2)
    zeros_buf = jnp.zeros((648, DP), f32)

    half = bs * NP
    name2_t0 = _sc_gather(obj_pad, class_p[:half], "sc_gather_names_t0")
    name2_t1 = _sc_gather(obj_pad, class_p[half:], "sc_gather_names_t1")
    h = _tc_init(name2_t0, states, w1a, ws2, cp, w2p, b2p, wih2)

    node_repr = None
    globs = []
    for t in range(T):
        xe = _sc_gather(h, origin[t], "sc_gather_edges")
        inp_e = _tc_edge(xe, et_cols, mk_cols, bmat, t)
        a = _sc_scatter(inp_e, dest[t], zeros_buf)
        if t == 0:
            t1_pack = (name2_t1, states, w1a, ws2, cp, w2p, b2p, wih2, whh2)
            node_repr, glob, h = _tc_update(a, h, vis_cols, wih1, whh1, bih1,
                                            bhh1, 0, T, N, None, t1_pack)
        else:
            node_repr, glob = _tc_update(a, h, vis_cols, wih1, whh1, bih1,
                                         bhh1, t, T, N, node_repr)
        globs.append(glob[:, 0, :D])

    global_repr = jnp.stack(globs, axis=1)
    return (node_repr, global_repr)


# speedup vs baseline: 1.0009x; 1.0009x over previous
import functools

import jax
import jax.numpy as jnp
from jax import lax
from jax.experimental import pallas as pl
from jax.experimental.pallas import tpu as pltpu
from jax.experimental.pallas import tpu_sc as plsc

D = 100
DP = 128
NT = 16
NSTATE = 16
NC, NS = 2, 16
NW = NC * NS
NP = 10240
HALF = NP
BLK = 512


def _sc_gather(table, idx, name):
    B = idx.shape[0]
    per_w = B // NW
    mesh = plsc.VectorSubcoreMesh(core_axis_name="c", subcore_axis_name="s")
    packed = table.dtype == jnp.bfloat16
    if packed:
        table = lax.bitcast_convert_type(
            table.reshape(table.shape[0], DP // 2, 2), jnp.int32)

    def _unpack(out):
        if packed:
            return lax.bitcast_convert_type(out, jnp.bfloat16).reshape(B, DP)
        return out
    W = table.shape[1]

    if per_w <= 128:
        @functools.partial(
            pl.kernel,
            mesh=mesh,
            out_type=jax.ShapeDtypeStruct((B, W), table.dtype),
            scratch_types=[
                pltpu.VMEM((per_w,), jnp.int32),
                pltpu.VMEM((per_w, W), table.dtype),
                pltpu.SemaphoreType.DMA,
            ],
            name=name,
        )
        def k(table_hbm, idx_hbm, out_hbm, idx_v, rows_v, sem):
            wid = lax.axis_index("s") * NC + lax.axis_index("c")
            base = wid * per_w
            pltpu.sync_copy(idx_hbm.at[pl.ds(base, per_w)], idx_v)
            pltpu.async_copy(table_hbm.at[idx_v], rows_v, sem).wait()
            pltpu.sync_copy(rows_v, out_hbm.at[pl.ds(base, per_w)])

        return _unpack(k(table, idx))

    assert per_w % 128 == 0
    nrow = per_w // 128
    nch = nrow
    nrow_p = (nrow + 7) // 8 * 8
    idx2d = jnp.zeros((NW, nrow_p, 128), jnp.int32).at[:, :nrow].set(
        idx.reshape(NW, nrow, 128)).reshape(NW * nrow_p, 128)

    @functools.partial(
        pl.kernel,
        mesh=mesh,
        out_type=jax.ShapeDtypeStruct((B, W), table.dtype),
        scratch_types=[
            pltpu.VMEM((nrow_p, 128), jnp.int32),
            pltpu.VMEM((128, W), table.dtype),
            pltpu.VMEM((128, W), table.dtype),
            pltpu.SemaphoreType.DMA,
            pltpu.SemaphoreType.DMA,
            pltpu.SemaphoreType.DMA,
            pltpu.SemaphoreType.DMA,
        ],
        name=name,
    )
    def kb(table_hbm, idx_hbm, out_hbm, idx_v, buf0, buf1, g0, g1, w0, w1):
        wid = lax.axis_index("s") * NC + lax.axis_index("c")
        base = wid * nrow
        pltpu.sync_copy(idx_hbm.at[pl.ds(wid * nrow_p, nrow_p)], idx_v)
        bufs, gsem, wsem = [buf0, buf1], [g0, g1], [w0, w1]
        gops = [None] * nch
        wops = [None] * nch

        def fire(ch):
            p = ch % 2
            gops[ch] = pltpu.async_copy(table_hbm.at[idx_v.at[ch]],
                                        bufs[p], gsem[p])

        fire(0)
        for ch in range(nch):
            p = ch % 2
            if ch + 1 < nch:
                if ch >= 1:
                    wops[ch - 1].wait()
                fire(ch + 1)
            gops[ch].wait()
            wops[ch] = pltpu.async_copy(
                bufs[p], out_hbm.at[pl.ds((base + ch) * 128, 128)], wsem[p])
        wops[nch - 2].wait()
        wops[nch - 1].wait()

    return _unpack(kb(table, idx2d))


def _sc_scatter(inp_e, dest, zeros_buf):
    E2 = dest.shape[0]
    ch = E2 // NS
    rows_per_tile = HALF // NS
    mesh = plsc.VectorSubcoreMesh(core_axis_name="c", subcore_axis_name="s")

    @functools.partial(
        pl.kernel,
        mesh=mesh,
        out_type=jax.ShapeDtypeStruct((2 * HALF, DP), jnp.float32),
        scratch_types=[
            pltpu.VMEM((ch,), jnp.int32),
            pltpu.VMEM((ch, DP), jnp.float32),
            pltpu.VMEM_SHARED((HALF + 8, DP), jnp.float32),
            pltpu.SemaphoreType.DMA,
        ],
        name="sc_scatter_add",
    )
    def k(inp_hbm, dest_hbm, zeros_hbm, out_hbm, idx_v, rows_v, acc, sem):
        c = lax.axis_index("c")
        s = lax.axis_index("s")
        base = c * HALF
        pltpu.sync_copy(zeros_hbm.at[pl.ds(0, rows_per_tile)],
                        acc.at[pl.ds(s * rows_per_tile, rows_per_tile)])

        @pl.when(s == 0)
        def _():
            pltpu.sync_copy(zeros_hbm.at[pl.ds(640, 8)], acc.at[pl.ds(HALF, 8)])

        plsc.subcore_barrier()
        pltpu.sync_copy(dest_hbm.at[pl.ds(s * ch, ch)], idx_v)
        for i in range(ch // 16):
            d = idx_v[pl.ds(i * 16, 16)]
            loc = d - base
            oob = (loc < 0) | (loc >= HALF)
            idx_v[pl.ds(i * 16, 16)] = jnp.where(oob, HALF, loc)
        pltpu.sync_copy(inp_hbm.at[pl.ds(s * ch, ch)], rows_v)
        pltpu.sync_copy(rows_v, acc.at[idx_v], add=True)
        plsc.subcore_barrier()
        pltpu.sync_copy(acc.at[pl.ds(s * rows_per_tile, rows_per_tile)],
                        out_hbm.at[pl.ds(base + s * rows_per_tile, rows_per_tile)])

    return k(inp_e, dest, zeros_buf)


def _f32dot(a, b):
    return jnp.dot(a, b, preferred_element_type=jnp.float32)


def _tc_prep(W_state, b_state2, W1, b12, W2, b22, edge_embed,
             g1wih, g1whh, g1bih2, g1bhh2, g2wih, g2whh, vis, etypes, masks):
    NBLK = NP // BLK
    n_real = vis.shape[2]

    def body(wst_r, bst_r, w1_r, b1_r, w2_r, b2_r, ee_r,
             w1i_r, w1h_r, b1i_r, b1h_r, w2i_r, w2h_r, vis_r, et_r, mk_r,
             w1a_o, ws2_o, cp_o, w2p_o, b2p_o, wih1_o, whh1_o, bih1_o,
             bhh1_o, wih2_o, whh2_o, bmat_o, visc_o, etc_o, mkc_o):
        i = pl.program_id(0)

        @pl.when(i == 0)
        def _weights():
            w1a = w1_r[:D, :]
            w1b = w1_r[D:, :]
            w1a_o[...] = jnp.zeros((DP, DP), jnp.float32)
            w1a_o[:D, :D] = w1a
            ws2_o[...] = jnp.zeros((NSTATE, DP), jnp.float32)
            ws2_o[:, :D] = _f32dot(wst_r[...], w1b)
            cp_o[...] = jnp.zeros((1, DP), jnp.float32)
            cp_o[:, :D] = _f32dot(bst_r[...], w1b) + b1_r[...]
            w2p_o[...] = jnp.zeros((DP, DP), jnp.float32)
            w2p_o[:D, :D] = w2_r[...]
            b2p_o[...] = jnp.zeros((1, DP), jnp.float32)
            b2p_o[:, :D] = b2_r[...]
            for src, dst in ((w1i_r, wih1_o), (w1h_r, whh1_o),
                             (w2i_r, wih2_o), (w2h_r, whh2_o)):
                dst[...] = jnp.zeros((DP, 3 * DP), jnp.float32)
                for g in range(3):
                    dst[:D, g * DP:g * DP + D] = src[:, g * D:(g + 1) * D]
            for src, dst in ((b1i_r, bih1_o), (b1h_r, bhh1_o)):
                dst[...] = jnp.zeros((1, 3 * DP), jnp.float32)
                for g in range(3):
                    dst[:, g * DP:g * DP + D] = src[:, g * D:(g + 1) * D]
            eet = jnp.transpose(ee_r[...].reshape(NT, D, D), (0, 2, 1))
            bmat_o[...] = jnp.zeros((NT * DP, DP), jnp.float32)
            for k in range(NT):
                bmat_o[k * DP:k * DP + D, :D] = eet[k]
            etm = et_r[...].reshape(4, -1)
            mkm = mk_r[...].reshape(4, -1)
            E = etm.shape[1]
            for b in range(2):
                for t in range(2):
                    r0 = t * 2 * E + b * E
                    etc_o[r0:r0 + E, :] = etm[2 * b + t].reshape(E, 1)
                    mkc_o[r0:r0 + E, :] = mkm[2 * b + t].reshape(E, 1)

        v = vis_r[...].reshape(4, BLK)
        col_ok = (lax.broadcasted_iota(jnp.int32, (4, BLK), 1)
                  + i * BLK) < n_real
        v = jnp.where(col_ok, v, 0.0)
        visc_o[...] = v.reshape(4, BLK, 1)

    E2 = 2 * etypes.shape[2]
    full = lambda *shape: pl.BlockSpec(shape, lambda i: tuple(0 for _ in shape))
    return pl.pallas_call(
        body,
        grid=(NBLK,),
        in_specs=[
            full(*W_state.shape), full(*b_state2.shape),
            full(*W1.shape), full(*b12.shape), full(*W2.shape),
            full(*b22.shape), full(*edge_embed.shape),
            full(*g1wih.shape), full(*g1whh.shape), full(*g1bih2.shape),
            full(*g1bhh2.shape), full(*g2wih.shape), full(*g2whh.shape),
            pl.BlockSpec((2, 2, BLK), lambda i: (0, 0, i)),
            full(*etypes.shape), full(*masks.shape),
        ],
        out_specs=[
            full(DP, DP), full(NSTATE, DP), full(1, DP),
            full(DP, DP), full(1, DP),
            full(DP, 3 * DP), full(DP, 3 * DP), full(1, 3 * DP),
            full(1, 3 * DP), full(DP, 3 * DP), full(DP, 3 * DP),
            full(NT * DP, DP),
            pl.BlockSpec((4, BLK, 1), lambda i: (0, i, 0)),
            full(2 * E2, 1), full(2 * E2, 1),
        ],
        out_shape=[
            jax.ShapeDtypeStruct((DP, DP), jnp.float32),
            jax.ShapeDtypeStruct((NSTATE, DP), jnp.float32),
            jax.ShapeDtypeStruct((1, DP), jnp.float32),
            jax.ShapeDtypeStruct((DP, DP), jnp.float32),
            jax.ShapeDtypeStruct((1, DP), jnp.float32),
            jax.ShapeDtypeStruct((DP, 3 * DP), jnp.float32),
            jax.ShapeDtypeStruct((DP, 3 * DP), jnp.float32),
            jax.ShapeDtypeStruct((1, 3 * DP), jnp.float32),
            jax.ShapeDtypeStruct((1, 3 * DP), jnp.float32),
            jax.ShapeDtypeStruct((DP, 3 * DP), jnp.float32),
            jax.ShapeDtypeStruct((DP, 3 * DP), jnp.float32),
            jax.ShapeDtypeStruct((NT * DP, DP), jnp.float32),
            jax.ShapeDtypeStruct((4, NP, 1), jnp.float32),
            jax.ShapeDtypeStruct((2 * E2, 1), jnp.int32),
            jax.ShapeDtypeStruct((2 * E2, 1), jnp.float32),
        ],
        name="tc_prep",
    )(W_state, b_state2, W1, b12, W2, b22, edge_embed,
      g1wih, g1whh, g1bih2, g1bhh2, g2wih, g2whh, vis, etypes, masks)


def _tc_init(name2_t0, states, w1a, ws2, cp, w2, b2, wih2):
    NBLK = NP // BLK

    def body(n0, s0, w1a_r, ws, c, w2r, b2r, wih, h0_ref):
        pre0 = (_f32dot(n0[...].astype(jnp.float32), w1a_r[...])
                + _f32dot(s0[...].reshape(BLK, NSTATE), ws[...]) + c[...])
        init0 = _f32dot(jax.nn.relu(pre0), w2r[...]) + b2r[...]
        gi = _f32dot(init0, wih[...])
        z = jax.nn.sigmoid(gi[:, DP:2 * DP])
        nn_ = jnp.tanh(gi[:, 2 * DP:])
        h0_ref[...] = (1.0 - z) * nn_

    row_spec = pl.BlockSpec((BLK, DP), lambda b, nb: (b * NBLK + nb, 0))
    return pl.pallas_call(
        body,
        grid=(2, NBLK),
        in_specs=[
            pl.BlockSpec((BLK, DP), lambda b, nb: (b * NBLK + nb, 0)),
            pl.BlockSpec((1, 1, BLK, NSTATE), lambda b, nb: (b, 0, nb, 0)),
            pl.BlockSpec((DP, DP), lambda b, nb: (0, 0)),
            pl.BlockSpec((NSTATE, DP), lambda b, nb: (0, 0)),
            pl.BlockSpec((1, DP), lambda b, nb: (0, 0)),
            pl.BlockSpec((DP, DP), lambda b, nb: (0, 0)),
            pl.BlockSpec((1, DP), lambda b, nb: (0, 0)),
            pl.BlockSpec((DP, 3 * DP), lambda b, nb: (0, 0)),
        ],
        out_specs=row_spec,
        out_shape=jax.ShapeDtypeStruct((2 * NP, DP), jnp.float32),
        name="tc_init",
    )(name2_t0, states, w1a, ws2, cp, w2, b2, wih2)


def _tc_edge(xe, et_cols, mk_cols, bmat, t_idx):
    E2 = xe.shape[0]
    EB = 256
    toff = t_idx * (E2 // EB)

    def body(xe_r, et_r, mk_r, b_r, out_r):
        xm = xe_r[...].astype(jnp.float32) * mk_r[...]
        xt = jnp.concatenate([xm] * NT, axis=1)
        kcol = lax.broadcasted_iota(jnp.int32, (EB, NT * DP), 1) // DP
        sel = (et_r[...] == kcol).astype(jnp.float32)
        out_r[...] = _f32dot(xt * sel, b_r[...])

    return pl.pallas_call(
        body,
        grid=(E2 // EB,),
        in_specs=[
            pl.BlockSpec((EB, DP), lambda i: (i, 0)),
            pl.BlockSpec((EB, 1), lambda i: (i + toff, 0)),
            pl.BlockSpec((EB, 1), lambda i: (i + toff, 0)),
            pl.BlockSpec((NT * DP, DP), lambda i: (0, 0)),
        ],
        out_specs=pl.BlockSpec((EB, DP), lambda i: (i, 0)),
        out_shape=jax.ShapeDtypeStruct((E2, DP), jnp.float32),
        name="tc_edge_matmul",
    )(xe, et_cols, mk_cols, bmat)


def _gru_gates(gi, gh, h):
    r = jax.nn.sigmoid(gi[:, :DP] + gh[:, :DP])
    z = jax.nn.sigmoid(gi[:, DP:2 * DP] + gh[:, DP:2 * DP])
    nn_ = jnp.tanh(gi[:, 2 * DP:] + r * gh[:, 2 * DP:])
    return (1.0 - z) * nn_ + z * h


def _tc_update(a, h, vis2, wih1, whh1, bih1, bhh1, t_idx, n_t, n_real,
               repr_buf=None, init_next=None, wih2=None, whh2=None):
    NBLK = NP // BLK
    bs = 2
    has_next = init_next is not None
    has_alias = repr_buf is not None

    def body(*refs):
        i = 7
        (a_r, h_r, vis_r, wih1_r, whh1_r, bih1_r, bhh1_r) = refs[:7]
        if has_alias:
            i += 1
        if has_next:
            (n1_r, s1_r, w1a_r, ws2_r, cp_r, w2p_r, b2p_r, wih2_r,
             whh2_r) = refs[i:i + 9]
            i += 9
        repr_r, glob_r = refs[i:i + 2]
        i += 2
        if has_next:
            h1_r = refs[i]
            i += 1
        accg, accv = refs[i:i + 2]
        nb = pl.program_id(1)
        h_ = h_r[...].astype(jnp.float32)
        gi = _f32dot(a_r[...], wih1_r[...]) + bih1_r[...]
        gh = _f32dot(h_, whh1_r[...]) + bhh1_r[...]
        feat = _gru_gates(gi, gh, h_)
        vis = vis_r[...].reshape(BLK, 1)
        rep = feat * vis
        repr_r[...] = rep[None, None, :, :D]
        ok = (lax.broadcasted_iota(jnp.int32, (BLK, 1), 0) + nb * BLK) < n_real
        part = jnp.sum(jnp.where(ok, rep * vis, 0.0), axis=0, keepdims=True)
        vpart = jnp.sum(vis) * jnp.ones((1, DP), jnp.float32)
        prev_g = jnp.where(nb == 0, 0.0, accg[...])
        prev_v = jnp.where(nb == 0, 0.0, accv[...])
        accg[...] = prev_g + part
        accv[...] = prev_v + vpart
        glob_r[...] = jnp.broadcast_to(
            (accg[...] / (accv[...] + 1e-9))[None], (1, 8, DP))
        if has_next:
            pre1 = (_f32dot(n1_r[...].astype(jnp.float32), w1a_r[...])
                    + _f32dot(s1_r[...].reshape(BLK, NSTATE), ws2_r[...])
                    + cp_r[...])
            init1 = _f32dot(jax.nn.relu(pre1), w2p_r[...]) + b2p_r[...]
            gi2 = _f32dot(init1, wih2_r[...])
            gh2 = _f32dot(h_, whh2_r[...])
            h1_r[...] = _gru_gates(gi2, gh2, h_)

    row_spec = pl.BlockSpec((BLK, DP), lambda b, nb: (b * NBLK + nb, 0))
    w_spec = pl.BlockSpec((DP, 3 * DP), lambda b, nb: (0, 0))
    b_spec = pl.BlockSpec((1, 3 * DP), lambda b, nb: (0, 0))
    in_specs = [row_spec, row_spec,
                pl.BlockSpec((1, BLK, 1), lambda b, nb: (2 * b + t_idx, nb, 0)),
                w_spec, w_spec, b_spec, b_spec]
    args = [a, h, vis2, wih1, whh1, bih1, bhh1]
    aliases = {}
    if has_alias:
        in_specs.append(pl.BlockSpec(memory_space=pl.ANY))
        args.append(repr_buf)
        aliases = {7: 0}
    if has_next:
        (name2_t1, states_raw, w1a, ws2, cp, w2p, b2p, wih2, whh2) = init_next
        in_specs += [
            pl.BlockSpec((BLK, DP), lambda b, nb: (b * NBLK + nb, 0)),
            pl.BlockSpec((1, 1, BLK, NSTATE), lambda b, nb: (b, 1, nb, 0)),
            pl.BlockSpec((DP, DP), lambda b, nb: (0, 0)),
            pl.BlockSpec((NSTATE, DP), lambda b, nb: (0, 0)),
            pl.BlockSpec((1, DP), lambda b, nb: (0, 0)),
            pl.BlockSpec((DP, DP), lambda b, nb: (0, 0)),
            pl.BlockSpec((1, DP), lambda b, nb: (0, 0)),
            w_spec, w_spec]
        args += [name2_t1, states_raw, w1a, ws2, cp, w2p, b2p, wih2, whh2]
    out_specs = [
        pl.BlockSpec((1, 1, BLK, D), lambda b, nb: (b, t_idx, nb, 0)),
        pl.BlockSpec((1, 8, DP), lambda b, nb: (b, 0, 0)),
    ]
    out_shape = [
        jax.ShapeDtypeStruct((bs, n_t, n_real, D), jnp.float32),
        jax.ShapeDtypeStruct((bs, 8, DP), jnp.float32),
    ]
    if has_next:
        out_specs.append(row_spec)
        out_shape.append(jax.ShapeDtypeStruct((bs * NP, DP), jnp.float32))

    return pl.pallas_call(
        body,
        grid=(bs, NBLK),
        in_specs=in_specs,
        out_specs=out_specs,
        out_shape=out_shape,
        input_output_aliases=aliases,
        scratch_shapes=[pltpu.VMEM((1, DP), jnp.float32),
                        pltpu.VMEM((1, DP), jnp.float32)],
        name="tc_update_t%d" % t_idx,
    )(*args)


def kernel(class_names, states, edge_values, edge_types, visibility, mask_edges,
           obj_emb, W_state, b_state, W1, b1, W2, b2, edge_embed,
           gru1_wih, gru1_whh, gru1_bih, gru1_bhh, gru2_wih, gru2_whh):
    bs, T, N = class_names.shape
    E = edge_values.shape[2]
    E2 = bs * E
    f32 = jnp.float32
    i32 = jnp.int32

    (w1a, ws2, cp, w2p, b2p, wih1, whh1, bih1, bhh1, wih2, whh2, bmat,
     vis_cols, et_cols, mk_cols) = _tc_prep(
        W_state, b_state.reshape(1, D), W1, b1.reshape(1, D), W2,
        b2.reshape(1, D), edge_embed, gru1_wih, gru1_whh,
        gru1_bih.reshape(1, 3 * D), gru1_bhh.reshape(1, 3 * D), gru2_wih,
        gru2_whh, visibility, edge_types.astype(i32), mask_edges)
    obj_pad = jnp.zeros((obj_emb.shape[0], DP), f32).at[:, :D].set(obj_emb)

    class_t = class_names.astype(i32).transpose(1, 0, 2)
    class_p = jnp.zeros((T, bs, NP), i32).at[:, :, :N].set(class_t).reshape(-1)
    offs = (jnp.arange(bs, dtype=i32) * NP)[None, :, None]
    ev = edge_values.astype(i32).transpose(1, 0, 2, 3)
    origin = (ev[:, :, :, 0] + offs).reshape(T, E2)
    dest = (ev[:, :, :, 1] + offs).reshape(T, E2)
    zeros_buf = jnp.zeros((648, DP), f32)

    half = bs * NP
    name2_t0 = _sc_gather(obj_pad, class_p[:half], "sc_gather_names_t0")
    name2_t1 = _sc_gather(obj_pad, class_p[half:], "sc_gather_names_t1")
    h = _tc_init(name2_t0, states, w1a, ws2, cp, w2p, b2p, wih2)

    node_repr = None
    globs = []
    for t in range(T):
        xe = _sc_gather(h, origin[t], "sc_gather_edges")
        inp_e = _tc_edge(xe, et_cols, mk_cols, bmat, t)
        a = _sc_scatter(inp_e, dest[t], zeros_buf)
        if t == 0:
            t1_pack = (name2_t1, states, w1a, ws2, cp, w2p, b2p, wih2, whh2)
            node_repr, glob, h = _tc_update(a, h, vis_cols, wih1, whh1, bih1,
                                            bhh1, 0, T, N, None, t1_pack)
        else:
            node_repr, glob = _tc_update(a, h, vis_cols, wih1, whh1, bih1,
                                         bhh1, t, T, N, node_repr)
        globs.append(glob[:, 0, :D])

    global_repr = jnp.stack(globs, axis=1)
    return (node_repr, global_repr)
